# baseline (speedup 1.0000x reference)
import functools

import jax
import jax.numpy as jnp
from jax import lax
from jax.experimental import pallas as pl
from jax.experimental.pallas import tpu as pltpu
from jax.experimental.pallas import tpu_sc as plsc

CB = 128


@functools.lru_cache(maxsize=None)
def _build(n_rows: int, vocab2: int):
    info = plsc.get_sparse_core_info()
    nw = info.num_cores * info.num_subcores
    n_units = n_rows // CB
    per_w = n_units // nw
    assert n_rows % CB == 0 and n_units % nw == 0 and per_w % 2 == 0
    n2 = per_w // 2

    mesh = plsc.VectorSubcoreMesh(core_axis_name="c", subcore_axis_name="s")

    @functools.partial(
        pl.kernel,
        mesh=mesh,
        out_type=jax.ShapeDtypeStruct((n_rows, 128), jnp.float32),
        scratch_types=[
            pltpu.VMEM((per_w, CB), jnp.int32),
            pltpu.VMEM((2, CB, 128), jnp.float32),
            pltpu.SemaphoreType.DMA,
            pltpu.SemaphoreType.DMA,
            pltpu.SemaphoreType.DMA,
            pltpu.SemaphoreType.DMA,
        ],
        compiler_params=pltpu.CompilerParams(use_tc_tiling_on_sc=True),
    )
    def gather_kernel(idx_hbm, t128_hbm, out_hbm, idx_v, rows_v, sg0, sg1, so0, so1):
        wid = lax.axis_index("s") * info.num_cores + lax.axis_index("c")
        ubase = wid * per_w
        sg = (sg0, sg1)
        so = (so0, so1)

        pltpu.sync_copy(idx_hbm.at[pl.ds(ubase, per_w)], idx_v)

        def gat(j, b):
            return pltpu.make_async_copy(
                t128_hbm.at[idx_v.at[j]], rows_v.at[b], sg[b]
            )

        def sto(j, b):
            return pltpu.make_async_copy(
                rows_v.at[b], out_hbm.at[pl.ds((ubase + j) * CB, CB)], so[b]
            )

        gat(0, 0).start()

        def body(j2, carry):
            j0 = 2 * j2
            j1 = j0 + 1
            gat(j0, 0).wait()
            sto(j0, 0).start()

            @pl.when(j2 > 0)
            def _():
                sto(j0 - 1, 1).wait()

            gat(j1, 1).start()
            gat(j1, 1).wait()
            sto(j1, 1).start()

            @pl.when(j2 < n2 - 1)
            def _():
                sto(j0, 0).wait()
                gat(j0 + 2, 0).start()

            return carry

        lax.fori_loop(0, n2, body, 0)
        sto(per_w - 2, 0).wait()
        sto(per_w - 1, 1).wait()

    return gather_kernel


def kernel(x, table):
    n_rows = x.shape[0] * x.shape[1]
    vocab, dim = table.shape
    xf = x.reshape(-1).astype(jnp.int32)
    idx2 = (xf >> 1).reshape(n_rows // CB, CB)
    t128 = table.reshape(vocab // 2, 2 * dim)
    fn = _build(n_rows, vocab // 2)
    o = fn(idx2, t128)
    odd = (xf & 1)[:, None] == 1
    out = jnp.where(odd, o[:, dim:], o[:, :dim])
    return out.reshape(x.shape + (dim,))
